# baseline (speedup 1.0000x reference)
import jax
import jax.numpy as jnp
from jax import lax
from jax.experimental import pallas as pl
from jax.experimental.pallas import tpu as pltpu
from jax.experimental.pallas import tpu_sc as plsc

N = 10000
E = 320000
IN_DIM = 128
H1, D1 = 8, 16

NC, NS, LANES = 2, 16, 16
NT = NC * NS
EPT = E // NT
C = 80
NCHUNK = 126
EPT_P = C * NCHUNK
NTRIP = NCHUNK // 3
NP = 10112
ROWS = NP // NS
ACC_W = 144
TAB_W = 144

_BLK = 1000


def _proj_kernel(x_ref, w_ref, tab_ref, er_ref):
    o = jnp.dot(x_ref[...], w_ref[...], preferred_element_type=jnp.float32)
    tab_ref[...] = o[:, :TAB_W]
    er_ref[...] = o[:, TAB_W:]


def _proj(x, Wcat):
    n = x.shape[0]
    k = Wcat.shape[1]
    return pl.pallas_call(
        _proj_kernel,
        grid=(n // _BLK,),
        in_specs=[
            pl.BlockSpec((_BLK, IN_DIM), lambda i: (i, 0)),
            pl.BlockSpec((IN_DIM, k), lambda i: (0, 0)),
        ],
        out_specs=[
            pl.BlockSpec((_BLK, TAB_W), lambda i: (i, 0)),
            pl.BlockSpec((_BLK, 16), lambda i: (i, 0)),
        ],
        out_shape=[
            jax.ShapeDtypeStruct((n, TAB_W), jnp.float32),
            jax.ShapeDtypeStruct((n, 16), jnp.float32),
        ],
    )(x, Wcat)


def _edge1_body(tab_hbm, er_hbm, src_hbm, dst_hbm, z_hbm, acc_out,
                srcA, dstA, feA, erA,
                srcB, dstB, feB, erB,
                srcC, dstC, feC, erC,
                acc_sh, gsA, gsB, gsC, ssA, ssB, ssC):
    cid = lax.axis_index("c")
    sid = lax.axis_index("s")
    wid = sid * NC + cid
    pltpu.sync_copy(z_hbm.at[pl.ds(sid * ROWS, ROWS)],
                    acc_sh.at[pl.ds(sid * ROWS, ROWS)])
    plsc.subcore_barrier()
    base0 = wid * EPT_P

    def load_idx(g, sv, dv):
        pltpu.sync_copy(src_hbm.at[pl.ds(base0 + g * C, C)], sv)
        pltpu.sync_copy(dst_hbm.at[pl.ds(base0 + g * C, C)], dv)

    def issue_gather(sv, dv, fe, er, sem):
        pltpu.async_copy(tab_hbm.at[sv], fe, sem)
        pltpu.async_copy(er_hbm.at[dv], er, sem)

    def wait_gather(sv, dv, fe, er, sem):
        pltpu.make_async_copy(tab_hbm.at[sv], fe, sem).wait()
        pltpu.make_async_copy(er_hbm.at[dv], er, sem).wait()

    def compute(fe, er):
        @pl.loop(0, C)
        def _edge(j):
            e = fe[j, pl.ds(128, 16)] + er[j, :]
            w = jnp.exp(jnp.maximum(e, 0.2 * e))
            fe[j, pl.ds(128, 16)] = w
            for h in range(H1):
                fe[j, pl.ds(h * D1, D1)] = fe[j, pl.ds(h * D1, D1)] * w[h]

    def issue_scatter(msg, dv, sem):
        pltpu.async_copy(msg, acc_sh.at[dv], sem, add=True)

    def wait_scatter(msg, dv, sem):
        pltpu.make_async_copy(msg, acc_sh.at[dv], sem).wait()

    load_idx(0, srcA, dstA)
    issue_gather(srcA, dstA, feA, erA, gsA)
    load_idx(1, srcB, dstB)
    issue_gather(srcB, dstB, feB, erB, gsB)

    @pl.loop(0, NTRIP)
    def _trip(k):
        g = 3 * k

        wait_gather(srcA, dstA, feA, erA, gsA)
        compute(feA, erA)
        issue_scatter(feA, dstA, ssA)

        @pl.when(k > 0)
        def _():
            wait_scatter(feC, dstC, ssC)
        load_idx(g + 2, srcC, dstC)
        issue_gather(srcC, dstC, feC, erC, gsC)

        wait_gather(srcB, dstB, feB, erB, gsB)
        compute(feB, erB)
        issue_scatter(feB, dstB, ssB)

        wait_scatter(feA, dstA, ssA)
        load_idx(g + 3, srcA, dstA)
        issue_gather(srcA, dstA, feA, erA, gsA)

        wait_gather(srcC, dstC, feC, erC, gsC)
        compute(feC, erC)
        issue_scatter(feC, dstC, ssC)

        wait_scatter(feB, dstB, ssB)
        load_idx(g + 4, srcB, dstB)
        issue_gather(srcB, dstB, feB, erB, gsB)

    wait_scatter(feC, dstC, ssC)
    wait_gather(srcA, dstA, feA, erA, gsA)
    wait_gather(srcB, dstB, feB, erB, gsB)
    plsc.subcore_barrier()
    pltpu.sync_copy(acc_sh.at[pl.ds(sid * ROWS, ROWS)],
                    acc_out.at[cid].at[pl.ds(sid * ROWS, ROWS)])


def _edge1(tab, er16p, src, dst, zeros144):
    mesh = plsc.VectorSubcoreMesh(core_axis_name="c", subcore_axis_name="s")
    return pl.kernel(
        _edge1_body,
        out_type=jax.ShapeDtypeStruct((NC, NP, ACC_W), jnp.float32),
        mesh=mesh,
        compiler_params=pltpu.CompilerParams(use_tc_tiling_on_sc=False),
        scratch_types=[
            pltpu.VMEM((C,), jnp.int32),
            pltpu.VMEM((C,), jnp.int32),
            pltpu.VMEM((C, TAB_W), jnp.float32),
            pltpu.VMEM((C, 16), jnp.float32),
            pltpu.VMEM((C,), jnp.int32),
            pltpu.VMEM((C,), jnp.int32),
            pltpu.VMEM((C, TAB_W), jnp.float32),
            pltpu.VMEM((C, 16), jnp.float32),
            pltpu.VMEM((C,), jnp.int32),
            pltpu.VMEM((C,), jnp.int32),
            pltpu.VMEM((C, TAB_W), jnp.float32),
            pltpu.VMEM((C, 16), jnp.float32),
            pltpu.VMEM_SHARED((NP, ACC_W), jnp.float32),
            pltpu.SemaphoreType.DMA,
            pltpu.SemaphoreType.DMA,
            pltpu.SemaphoreType.DMA,
            pltpu.SemaphoreType.DMA,
            pltpu.SemaphoreType.DMA,
            pltpu.SemaphoreType.DMA,
        ],
    )(tab, er16p, src, dst, zeros144)


def _fin1_kernel(acc_ref, rep_ref, b1_ref, w2_ref, rw2_ref, f2_ref, hr_ref):
    acc = acc_ref[0] + acc_ref[1]
    numer = acc[:, :128]
    den = jnp.dot(acc[:, 128:144], rep_ref[...],
                  preferred_element_type=jnp.float32)
    rst = numer / (den + 1e-9) + b1_ref[...]
    h = jnp.where(rst > 0, rst, jnp.exp(rst) - 1.0)
    f2_ref[...] = jnp.dot(h, w2_ref[...], preferred_element_type=jnp.float32)
    hr_ref[...] = jnp.dot(h, rw2_ref[...], preferred_element_type=jnp.float32)


def _fin1(acc, REP, b1r, W2_16, RW2_16):
    return pl.pallas_call(
        _fin1_kernel,
        grid=(N // _BLK,),
        in_specs=[
            pl.BlockSpec((NC, _BLK, ACC_W), lambda i: (0, i, 0)),
            pl.BlockSpec((16, 128), lambda i: (0, 0)),
            pl.BlockSpec((1, 128), lambda i: (0, 0)),
            pl.BlockSpec((128, 16), lambda i: (0, 0)),
            pl.BlockSpec((128, 16), lambda i: (0, 0)),
        ],
        out_specs=[
            pl.BlockSpec((_BLK, 16), lambda i: (i, 0)),
            pl.BlockSpec((_BLK, 16), lambda i: (i, 0)),
        ],
        out_shape=[
            jax.ShapeDtypeStruct((N, 16), jnp.float32),
            jax.ShapeDtypeStruct((N, 16), jnp.float32),
        ],
    )(acc, REP, b1r, W2_16, RW2_16)


def _edge2_body(f2_hbm, src_hbm, dst_hbm, z_hbm, al2_hbm, ar2_hbm, acc_out,
                srcA, dstA, gsA_v, gdA_v, outA,
                srcB, dstB, gsB_v, gdB_v, outB,
                srcC, dstC, gsC_v, gdC_v, outC,
                al2_v, ar2_v, acc_sh, gsA, gsB, gsC, ssA, ssB, ssC):
    cid = lax.axis_index("c")
    sid = lax.axis_index("s")
    wid = sid * NC + cid
    pltpu.sync_copy(al2_hbm, al2_v)
    pltpu.sync_copy(ar2_hbm, ar2_v)
    pltpu.sync_copy(z_hbm.at[pl.ds(sid * ROWS, ROWS)],
                    acc_sh.at[pl.ds(sid * ROWS, ROWS)])
    plsc.subcore_barrier()
    base0 = wid * EPT_P
    iota = lax.iota(jnp.int32, LANES)
    m0 = jnp.where(iota == 0, 1.0, 0.0)
    m1 = jnp.where(iota == 1, 1.0, 0.0)
    al2v = al2_v[...]
    ar2v = ar2_v[...]

    def load_idx(g, sv, dv):
        pltpu.sync_copy(src_hbm.at[pl.ds(base0 + g * C, C)], sv)
        pltpu.sync_copy(dst_hbm.at[pl.ds(base0 + g * C, C)], dv)

    def issue_gather(sv, dv, gs_v, gd_v, sem):
        pltpu.async_copy(f2_hbm.at[sv], gs_v, sem)
        pltpu.async_copy(f2_hbm.at[dv], gd_v, sem)

    def wait_gather(sv, dv, gs_v, gd_v, sem):
        pltpu.make_async_copy(f2_hbm.at[sv], gs_v, sem).wait()
        pltpu.make_async_copy(f2_hbm.at[dv], gd_v, sem).wait()

    def compute(gs_v, gd_v, out_v):
        @pl.loop(0, C)
        def _edge(j):
            gs = gs_v[j, :]
            gd = gd_v[j, :]
            e = gs * al2v + gd * ar2v
            w = jnp.exp(jnp.maximum(e, 0.2 * e))
            out_v[j, :] = w * (gs * m0 + m1)

    def issue_scatter(out_v, dv, sem):
        pltpu.async_copy(out_v, acc_sh.at[dv], sem, add=True)

    def wait_scatter(out_v, dv, sem):
        pltpu.make_async_copy(out_v, acc_sh.at[dv], sem).wait()

    load_idx(0, srcA, dstA)
    issue_gather(srcA, dstA, gsA_v, gdA_v, gsA)
    load_idx(1, srcB, dstB)
    issue_gather(srcB, dstB, gsB_v, gdB_v, gsB)

    @pl.loop(0, NTRIP)
    def _trip(k):
        g = 3 * k

        wait_gather(srcA, dstA, gsA_v, gdA_v, gsA)
        compute(gsA_v, gdA_v, outA)
        issue_scatter(outA, dstA, ssA)

        @pl.when(k > 0)
        def _():
            wait_scatter(outC, dstC, ssC)
        load_idx(g + 2, srcC, dstC)
        issue_gather(srcC, dstC, gsC_v, gdC_v, gsC)

        wait_gather(srcB, dstB, gsB_v, gdB_v, gsB)
        compute(gsB_v, gdB_v, outB)
        issue_scatter(outB, dstB, ssB)

        wait_scatter(outA, dstA, ssA)
        load_idx(g + 3, srcA, dstA)
        issue_gather(srcA, dstA, gsA_v, gdA_v, gsA)

        wait_gather(srcC, dstC, gsC_v, gdC_v, gsC)
        compute(gsC_v, gdC_v, outC)
        issue_scatter(outC, dstC, ssC)

        wait_scatter(outB, dstB, ssB)
        load_idx(g + 4, srcB, dstB)
        issue_gather(srcB, dstB, gsB_v, gdB_v, gsB)

    wait_scatter(outC, dstC, ssC)
    wait_gather(srcA, dstA, gsA_v, gdA_v, gsA)
    wait_gather(srcB, dstB, gsB_v, gdB_v, gsB)
    plsc.subcore_barrier()
    pltpu.sync_copy(acc_sh.at[pl.ds(sid * ROWS, ROWS)],
                    acc_out.at[cid].at[pl.ds(sid * ROWS, ROWS)])


def _edge2(f2p, src, dst, zeros16, al2b, ar2b):
    mesh = plsc.VectorSubcoreMesh(core_axis_name="c", subcore_axis_name="s")
    return pl.kernel(
        _edge2_body,
        out_type=jax.ShapeDtypeStruct((NC, NP, 16), jnp.float32),
        mesh=mesh,
        compiler_params=pltpu.CompilerParams(use_tc_tiling_on_sc=False),
        scratch_types=[
            pltpu.VMEM((C,), jnp.int32),
            pltpu.VMEM((C,), jnp.int32),
            pltpu.VMEM((C, 16), jnp.float32),
            pltpu.VMEM((C, 16), jnp.float32),
            pltpu.VMEM((C, 16), jnp.float32),
            pltpu.VMEM((C,), jnp.int32),
            pltpu.VMEM((C,), jnp.int32),
            pltpu.VMEM((C, 16), jnp.float32),
            pltpu.VMEM((C, 16), jnp.float32),
            pltpu.VMEM((C, 16), jnp.float32),
            pltpu.VMEM((C,), jnp.int32),
            pltpu.VMEM((C,), jnp.int32),
            pltpu.VMEM((C, 16), jnp.float32),
            pltpu.VMEM((C, 16), jnp.float32),
            pltpu.VMEM((C, 16), jnp.float32),
            pltpu.VMEM((LANES,), jnp.float32),
            pltpu.VMEM((LANES,), jnp.float32),
            pltpu.VMEM_SHARED((NP, 16), jnp.float32),
            pltpu.SemaphoreType.DMA,
            pltpu.SemaphoreType.DMA,
            pltpu.SemaphoreType.DMA,
            pltpu.SemaphoreType.DMA,
            pltpu.SemaphoreType.DMA,
            pltpu.SemaphoreType.DMA,
        ],
    )(f2p, src, dst, zeros16, al2b, ar2b)


def _fin2_kernel(acc_ref, hr_ref, b2_ref, o_ref):
    acc = acc_ref[0] + acc_ref[1]
    numer = acc[:, 0:1]
    den = acc[:, 1:2]
    o_ref[...] = numer / (den + 1e-9) + hr_ref[:, 0:1] + b2_ref[0, 0]


def _fin2(acc2, hr, b2r):
    return pl.pallas_call(
        _fin2_kernel,
        grid=(N // _BLK,),
        in_specs=[
            pl.BlockSpec((NC, _BLK, 16), lambda i: (0, i, 0)),
            pl.BlockSpec((_BLK, 16), lambda i: (i, 0)),
            pl.BlockSpec((1, 1), lambda i: (0, 0)),
        ],
        out_specs=pl.BlockSpec((_BLK, 1), lambda i: (i, 0)),
        out_shape=jax.ShapeDtypeStruct((N, 1), jnp.float32),
    )(acc2, hr, b2r)


def _head_matrix(a):
    k = jnp.arange(H1 * D1)
    M = jnp.zeros((H1 * D1, 16), jnp.float32)
    return M.at[k, k // D1].set(a.reshape(H1 * D1))


def _pad_edges(v, fill):
    v2 = v.reshape(NT, EPT)
    v2 = jnp.pad(v2, ((0, 0), (0, EPT_P - EPT)), constant_values=fill)
    return jnp.pad(v2.reshape(-1), (0, 2 * C), constant_values=fill)


def kernel(features, edge_index, W1, al1, ar1, b1, W2, al2, ar2, rw2, b2):
    src = edge_index[0]
    dst = edge_index[1]

    Wcat = jnp.concatenate(
        [W1, W1 @ _head_matrix(al1), W1 @ _head_matrix(ar1)], axis=1)
    k128 = jnp.arange(128)
    REP = jnp.zeros((16, 128), jnp.float32).at[k128 // D1, k128].set(1.0)
    b1r = b1.reshape(1, 128)
    W2_16 = jnp.tile(W2, (1, 16))
    RW2_16 = jnp.tile(rw2, (1, 16))
    al2b = jnp.broadcast_to(al2.reshape(1), (LANES,))
    ar2b = jnp.broadcast_to(ar2.reshape(1), (LANES,))
    zeros144 = jnp.zeros((NP, ACC_W), jnp.float32)
    zeros16 = jnp.zeros((NP, 16), jnp.float32)
    b2r = b2.reshape(1, 1)
    src_p = _pad_edges(src, 0)
    dst_p = _pad_edges(dst, N)

    tab, er16 = _proj(features, Wcat)
    er16p = jnp.concatenate([er16, jnp.zeros((NP - N, 16), jnp.float32)])
    acc = _edge1(tab, er16p, src_p, dst_p, zeros144)
    f2, hr = _fin1(acc, REP, b1r, W2_16, RW2_16)

    f2p = jnp.concatenate([f2, jnp.zeros((NP - N, 16), jnp.float32)])
    acc2 = _edge2(f2p, src_p, dst_p, zeros16, al2b, ar2b)
    return _fin2(acc2, hr, b2r)


# speedup vs baseline: 1.1785x; 1.1785x over previous
import jax
import jax.numpy as jnp
from jax import lax
from jax.experimental import pallas as pl
from jax.experimental.pallas import tpu as pltpu
from jax.experimental.pallas import tpu_sc as plsc

N = 10000
E = 320000
IN_DIM = 128
H1, D1 = 8, 16

NC, NS, LANES = 2, 16, 16
NT = NC * NS
EPT = E // NT
C = 80
NCHUNK = 126
EPT_P = C * NCHUNK
NTRIP = NCHUNK // 3
C2 = 120
NCHUNK2 = EPT_P // C2
NTRIP2 = NCHUNK2 // 3
NP = 10112
ROWS = NP // NS
ACC_W = 144
TAB_W = 144

_BLK = 1000


def _proj_kernel(x_ref, w_ref, tab_ref, er_ref):
    o = jnp.dot(x_ref[...], w_ref[...], preferred_element_type=jnp.float32)
    tab_ref[...] = o[:, :TAB_W]
    er_ref[...] = o[:, TAB_W:]


def _proj(x, Wcat):
    n = x.shape[0]
    k = Wcat.shape[1]
    return pl.pallas_call(
        _proj_kernel,
        grid=(n // _BLK,),
        in_specs=[
            pl.BlockSpec((_BLK, IN_DIM), lambda i: (i, 0)),
            pl.BlockSpec((IN_DIM, k), lambda i: (0, 0)),
        ],
        out_specs=[
            pl.BlockSpec((_BLK, TAB_W), lambda i: (i, 0)),
            pl.BlockSpec((_BLK, 16), lambda i: (i, 0)),
        ],
        out_shape=[
            jax.ShapeDtypeStruct((n, TAB_W), jnp.float32),
            jax.ShapeDtypeStruct((n, 16), jnp.float32),
        ],
    )(x, Wcat)


def _edge1_body(tab_hbm, er_hbm, src_hbm, dst_hbm, z_hbm, acc_out,
                srcA, dstA, feA, erA,
                srcB, dstB, feB, erB,
                srcC, dstC, feC, erC,
                acc_sh, gsA, gsB, gsC, ssA, ssB, ssC):
    cid = lax.axis_index("c")
    sid = lax.axis_index("s")
    wid = sid * NC + cid
    pltpu.sync_copy(z_hbm.at[pl.ds(sid * ROWS, ROWS)],
                    acc_sh.at[pl.ds(sid * ROWS, ROWS)])
    plsc.subcore_barrier()
    base0 = wid * EPT_P

    def load_idx(g, sv, dv, sem):
        a = pltpu.async_copy(src_hbm.at[pl.ds(base0 + g * C, C)], sv, sem)
        b = pltpu.async_copy(dst_hbm.at[pl.ds(base0 + g * C, C)], dv, sem)
        a.wait()
        b.wait()

    def issue_gather(sv, dv, fe, er, sem):
        pltpu.async_copy(tab_hbm.at[sv], fe, sem)
        pltpu.async_copy(er_hbm.at[dv], er, sem)

    def wait_gather(sv, dv, fe, er, sem):
        pltpu.make_async_copy(tab_hbm.at[sv], fe, sem).wait()
        pltpu.make_async_copy(er_hbm.at[dv], er, sem).wait()

    def compute(fe, er):
        @pl.loop(0, C)
        def _edge(j):
            e = fe[j, pl.ds(128, 16)] + er[j, :]
            w = jnp.exp(jnp.maximum(e, 0.2 * e))
            fe[j, pl.ds(128, 16)] = w
            for h in range(H1):
                fe[j, pl.ds(h * D1, D1)] = fe[j, pl.ds(h * D1, D1)] * w[h]

    def issue_scatter(msg, dv, sem):
        pltpu.async_copy(msg, acc_sh.at[dv], sem, add=True)

    def wait_scatter(msg, dv, sem):
        pltpu.make_async_copy(msg, acc_sh.at[dv], sem).wait()

    load_idx(0, srcA, dstA, gsA)
    issue_gather(srcA, dstA, feA, erA, gsA)
    load_idx(1, srcB, dstB, gsB)
    issue_gather(srcB, dstB, feB, erB, gsB)

    @pl.loop(0, NTRIP)
    def _trip(k):
        g = 3 * k

        wait_gather(srcA, dstA, feA, erA, gsA)
        compute(feA, erA)
        issue_scatter(feA, dstA, ssA)

        @pl.when(k > 0)
        def _():
            wait_scatter(feC, dstC, ssC)
        load_idx(g + 2, srcC, dstC, gsC)
        issue_gather(srcC, dstC, feC, erC, gsC)

        wait_gather(srcB, dstB, feB, erB, gsB)
        compute(feB, erB)
        issue_scatter(feB, dstB, ssB)

        wait_scatter(feA, dstA, ssA)
        load_idx(g + 3, srcA, dstA, gsA)
        issue_gather(srcA, dstA, feA, erA, gsA)

        wait_gather(srcC, dstC, feC, erC, gsC)
        compute(feC, erC)
        issue_scatter(feC, dstC, ssC)

        wait_scatter(feB, dstB, ssB)
        load_idx(g + 4, srcB, dstB, gsB)
        issue_gather(srcB, dstB, feB, erB, gsB)

    wait_scatter(feC, dstC, ssC)
    wait_gather(srcA, dstA, feA, erA, gsA)
    wait_gather(srcB, dstB, feB, erB, gsB)
    plsc.subcore_barrier()
    pltpu.sync_copy(acc_sh.at[pl.ds(sid * ROWS, ROWS)],
                    acc_out.at[cid].at[pl.ds(sid * ROWS, ROWS)])


def _edge1(tab, er16p, src, dst, zeros144):
    mesh = plsc.VectorSubcoreMesh(core_axis_name="c", subcore_axis_name="s")
    return pl.kernel(
        _edge1_body,
        out_type=jax.ShapeDtypeStruct((NC, NP, ACC_W), jnp.float32),
        mesh=mesh,
        compiler_params=pltpu.CompilerParams(use_tc_tiling_on_sc=False),
        scratch_types=[
            pltpu.VMEM((C,), jnp.int32),
            pltpu.VMEM((C,), jnp.int32),
            pltpu.VMEM((C, TAB_W), jnp.float32),
            pltpu.VMEM((C, 16), jnp.float32),
            pltpu.VMEM((C,), jnp.int32),
            pltpu.VMEM((C,), jnp.int32),
            pltpu.VMEM((C, TAB_W), jnp.float32),
            pltpu.VMEM((C, 16), jnp.float32),
            pltpu.VMEM((C,), jnp.int32),
            pltpu.VMEM((C,), jnp.int32),
            pltpu.VMEM((C, TAB_W), jnp.float32),
            pltpu.VMEM((C, 16), jnp.float32),
            pltpu.VMEM_SHARED((NP, ACC_W), jnp.float32),
            pltpu.SemaphoreType.DMA,
            pltpu.SemaphoreType.DMA,
            pltpu.SemaphoreType.DMA,
            pltpu.SemaphoreType.DMA,
            pltpu.SemaphoreType.DMA,
            pltpu.SemaphoreType.DMA,
        ],
    )(tab, er16p, src, dst, zeros144)


def _fin1_kernel(acc_ref, rep_ref, b1_ref, w2_ref, rw2_ref, f2_ref, hr_ref):
    acc = acc_ref[0] + acc_ref[1]
    numer = acc[:, :128]
    den = jnp.dot(acc[:, 128:144], rep_ref[...],
                  preferred_element_type=jnp.float32)
    rst = numer / (den + 1e-9) + b1_ref[...]
    h = jnp.where(rst > 0, rst, jnp.exp(rst) - 1.0)
    f2_ref[...] = jnp.dot(h, w2_ref[...], preferred_element_type=jnp.float32)
    hr_ref[...] = jnp.dot(h, rw2_ref[...], preferred_element_type=jnp.float32)


def _fin1(acc, REP, b1r, W2_16, RW2_16):
    return pl.pallas_call(
        _fin1_kernel,
        grid=(N // _BLK,),
        in_specs=[
            pl.BlockSpec((NC, _BLK, ACC_W), lambda i: (0, i, 0)),
            pl.BlockSpec((16, 128), lambda i: (0, 0)),
            pl.BlockSpec((1, 128), lambda i: (0, 0)),
            pl.BlockSpec((128, 16), lambda i: (0, 0)),
            pl.BlockSpec((128, 16), lambda i: (0, 0)),
        ],
        out_specs=[
            pl.BlockSpec((_BLK, 16), lambda i: (i, 0)),
            pl.BlockSpec((_BLK, 16), lambda i: (i, 0)),
        ],
        out_shape=[
            jax.ShapeDtypeStruct((N, 16), jnp.float32),
            jax.ShapeDtypeStruct((N, 16), jnp.float32),
        ],
    )(acc, REP, b1r, W2_16, RW2_16)


def _edge2_body(f2_hbm, src_hbm, dst_hbm, z_hbm, al2_hbm, ar2_hbm, acc_out,
                srcA, dstA, gsA_v, gdA_v, outA,
                srcB, dstB, gsB_v, gdB_v, outB,
                srcC, dstC, gsC_v, gdC_v, outC,
                al2_v, ar2_v, acc_sh, gsA, gsB, gsC, ssA, ssB, ssC):
    cid = lax.axis_index("c")
    sid = lax.axis_index("s")
    wid = sid * NC + cid
    pltpu.sync_copy(al2_hbm, al2_v)
    pltpu.sync_copy(ar2_hbm, ar2_v)
    pltpu.sync_copy(z_hbm.at[pl.ds(sid * ROWS, ROWS)],
                    acc_sh.at[pl.ds(sid * ROWS, ROWS)])
    plsc.subcore_barrier()
    base0 = wid * EPT_P
    iota = lax.iota(jnp.int32, LANES)
    m0 = jnp.where(iota == 0, 1.0, 0.0)
    m1 = jnp.where(iota == 1, 1.0, 0.0)
    al2v = al2_v[...]
    ar2v = ar2_v[...]

    def load_idx(g, sv, dv, sem):
        a = pltpu.async_copy(src_hbm.at[pl.ds(base0 + g * C2, C2)], sv, sem)
        b = pltpu.async_copy(dst_hbm.at[pl.ds(base0 + g * C2, C2)], dv, sem)
        a.wait()
        b.wait()

    def issue_gather(sv, dv, gs_v, gd_v, sem):
        pltpu.async_copy(f2_hbm.at[sv], gs_v, sem)
        pltpu.async_copy(f2_hbm.at[dv], gd_v, sem)

    def wait_gather(sv, dv, gs_v, gd_v, sem):
        pltpu.make_async_copy(f2_hbm.at[sv], gs_v, sem).wait()
        pltpu.make_async_copy(f2_hbm.at[dv], gd_v, sem).wait()

    def compute(gs_v, gd_v, out_v):
        @pl.loop(0, C2)
        def _edge(j):
            gs = gs_v[j, :]
            gd = gd_v[j, :]
            e = gs * al2v + gd * ar2v
            w = jnp.exp(jnp.maximum(e, 0.2 * e))
            out_v[j, :] = w * (gs * m0 + m1)

    def issue_scatter(out_v, dv, sem):
        pltpu.async_copy(out_v, acc_sh.at[dv], sem, add=True)

    def wait_scatter(out_v, dv, sem):
        pltpu.make_async_copy(out_v, acc_sh.at[dv], sem).wait()

    load_idx(0, srcA, dstA, gsA)
    issue_gather(srcA, dstA, gsA_v, gdA_v, gsA)
    load_idx(1, srcB, dstB, gsB)
    issue_gather(srcB, dstB, gsB_v, gdB_v, gsB)

    @pl.loop(0, NTRIP2)
    def _trip(k):
        g = 3 * k

        wait_gather(srcA, dstA, gsA_v, gdA_v, gsA)
        compute(gsA_v, gdA_v, outA)
        issue_scatter(outA, dstA, ssA)

        @pl.when(k > 0)
        def _():
            wait_scatter(outC, dstC, ssC)
        load_idx(g + 2, srcC, dstC, gsC)
        issue_gather(srcC, dstC, gsC_v, gdC_v, gsC)

        wait_gather(srcB, dstB, gsB_v, gdB_v, gsB)
        compute(gsB_v, gdB_v, outB)
        issue_scatter(outB, dstB, ssB)

        wait_scatter(outA, dstA, ssA)
        load_idx(g + 3, srcA, dstA, gsA)
        issue_gather(srcA, dstA, gsA_v, gdA_v, gsA)

        wait_gather(srcC, dstC, gsC_v, gdC_v, gsC)
        compute(gsC_v, gdC_v, outC)
        issue_scatter(outC, dstC, ssC)

        wait_scatter(outB, dstB, ssB)
        load_idx(g + 4, srcB, dstB, gsB)
        issue_gather(srcB, dstB, gsB_v, gdB_v, gsB)

    wait_scatter(outC, dstC, ssC)
    wait_gather(srcA, dstA, gsA_v, gdA_v, gsA)
    wait_gather(srcB, dstB, gsB_v, gdB_v, gsB)
    plsc.subcore_barrier()
    pltpu.sync_copy(acc_sh.at[pl.ds(sid * ROWS, ROWS)],
                    acc_out.at[cid].at[pl.ds(sid * ROWS, ROWS)])


def _edge2(f2p, src, dst, zeros16, al2b, ar2b):
    mesh = plsc.VectorSubcoreMesh(core_axis_name="c", subcore_axis_name="s")
    return pl.kernel(
        _edge2_body,
        out_type=jax.ShapeDtypeStruct((NC, NP, 16), jnp.float32),
        mesh=mesh,
        compiler_params=pltpu.CompilerParams(use_tc_tiling_on_sc=False),
        scratch_types=[
            pltpu.VMEM((C2,), jnp.int32),
            pltpu.VMEM((C2,), jnp.int32),
            pltpu.VMEM((C2, 16), jnp.float32),
            pltpu.VMEM((C2, 16), jnp.float32),
            pltpu.VMEM((C2, 16), jnp.float32),
            pltpu.VMEM((C2,), jnp.int32),
            pltpu.VMEM((C2,), jnp.int32),
            pltpu.VMEM((C2, 16), jnp.float32),
            pltpu.VMEM((C2, 16), jnp.float32),
            pltpu.VMEM((C2, 16), jnp.float32),
            pltpu.VMEM((C2,), jnp.int32),
            pltpu.VMEM((C2,), jnp.int32),
            pltpu.VMEM((C2, 16), jnp.float32),
            pltpu.VMEM((C2, 16), jnp.float32),
            pltpu.VMEM((C2, 16), jnp.float32),
            pltpu.VMEM((LANES,), jnp.float32),
            pltpu.VMEM((LANES,), jnp.float32),
            pltpu.VMEM_SHARED((NP, 16), jnp.float32),
            pltpu.SemaphoreType.DMA,
            pltpu.SemaphoreType.DMA,
            pltpu.SemaphoreType.DMA,
            pltpu.SemaphoreType.DMA,
            pltpu.SemaphoreType.DMA,
            pltpu.SemaphoreType.DMA,
        ],
    )(f2p, src, dst, zeros16, al2b, ar2b)


def _fin2_kernel(acc_ref, hr_ref, b2_ref, o_ref):
    acc = acc_ref[0] + acc_ref[1]
    numer = acc[:, 0:1]
    den = acc[:, 1:2]
    o_ref[...] = numer / (den + 1e-9) + hr_ref[:, 0:1] + b2_ref[0, 0]


def _fin2(acc2, hr, b2r):
    return pl.pallas_call(
        _fin2_kernel,
        grid=(N // _BLK,),
        in_specs=[
            pl.BlockSpec((NC, _BLK, 16), lambda i: (0, i, 0)),
            pl.BlockSpec((_BLK, 16), lambda i: (i, 0)),
            pl.BlockSpec((1, 1), lambda i: (0, 0)),
        ],
        out_specs=pl.BlockSpec((_BLK, 1), lambda i: (i, 0)),
        out_shape=jax.ShapeDtypeStruct((N, 1), jnp.float32),
    )(acc2, hr, b2r)


def _head_matrix(a):
    k = jnp.arange(H1 * D1)
    M = jnp.zeros((H1 * D1, 16), jnp.float32)
    return M.at[k, k // D1].set(a.reshape(H1 * D1))


def _pad_edges(v, fill):
    v2 = v.reshape(NT, EPT)
    v2 = jnp.pad(v2, ((0, 0), (0, EPT_P - EPT)), constant_values=fill)
    return jnp.pad(v2.reshape(-1), (0, 2 * C2), constant_values=fill)


def kernel(features, edge_index, W1, al1, ar1, b1, W2, al2, ar2, rw2, b2):
    src = edge_index[0]
    dst = edge_index[1]

    Wcat = jnp.concatenate(
        [W1, W1 @ _head_matrix(al1), W1 @ _head_matrix(ar1)], axis=1)
    k128 = jnp.arange(128)
    REP = jnp.zeros((16, 128), jnp.float32).at[k128 // D1, k128].set(1.0)
    b1r = b1.reshape(1, 128)
    W2_16 = jnp.tile(W2, (1, 16))
    RW2_16 = jnp.tile(rw2, (1, 16))
    al2b = jnp.broadcast_to(al2.reshape(1), (LANES,))
    ar2b = jnp.broadcast_to(ar2.reshape(1), (LANES,))
    zeros144 = jnp.zeros((NP, ACC_W), jnp.float32)
    zeros16 = jnp.zeros((NP, 16), jnp.float32)
    b2r = b2.reshape(1, 1)
    src_p = _pad_edges(src, 0)
    dst_p = _pad_edges(dst, N)

    tab, er16 = _proj(features, Wcat)
    er16p = jnp.concatenate([er16, jnp.zeros((NP - N, 16), jnp.float32)])
    acc = _edge1(tab, er16p, src_p, dst_p, zeros144)
    f2, hr = _fin1(acc, REP, b1r, W2_16, RW2_16)

    f2p = jnp.concatenate([f2, jnp.zeros((NP - N, 16), jnp.float32)])
    acc2 = _edge2(f2p, src_p, dst_p, zeros16, al2b, ar2b)
    return _fin2(acc2, hr, b2r)
